# baseline (device time: 200263 ns/iter reference)
import jax
import jax.numpy as jnp
from jax import lax
from jax.experimental import pallas as pl
from jax.experimental.pallas import tpu as pltpu

N_DEV = 8


def kernel(x, W1, W2):
    m, _ = x.shape
    d = W1.shape[1]
    n_out = W2.shape[1]
    chunk = m // N_DEV

    def body(x_ref, w1_ref, w2_ref, out_ref,
             h_ref, hfull_ref, recv_ref,
             rs_send_sem, rs_recv_sems, ag_send_sem, ag_recv_sems):
        my = lax.axis_index("i")
        left = lax.rem(my + N_DEV - 1, N_DEV)
        right = lax.rem(my + 1, N_DEV)

        barrier_sem = pltpu.get_barrier_semaphore()
        for nbr in (left, right):
            pl.semaphore_signal(
                barrier_sem, inc=1,
                device_id=(nbr,), device_id_type=pl.DeviceIdType.MESH,
            )
        pl.semaphore_wait(barrier_sem, 2)

        h_ref[...] = jnp.dot(x_ref[...], w1_ref[...],
                             preferred_element_type=jnp.float32)

        for s in range(N_DEV - 1):
            c_send = lax.rem(my - s + 2 * N_DEV, N_DEV)
            c_recv = lax.rem(my - s - 1 + 2 * N_DEV, N_DEV)
            rdma = pltpu.make_async_remote_copy(
                src_ref=h_ref.at[pl.ds(c_send * chunk, chunk), :],
                dst_ref=recv_ref.at[s],
                send_sem=rs_send_sem,
                recv_sem=rs_recv_sems.at[s],
                device_id=(right,),
                device_id_type=pl.DeviceIdType.MESH,
            )
            rdma.start()
            rdma.wait()
            h_ref[pl.ds(c_recv * chunk, chunk), :] = (
                h_ref[pl.ds(c_recv * chunk, chunk), :] + recv_ref[s]
            )
        own = lax.rem(my + 1, N_DEV)
        hfull_ref[pl.ds(own * chunk, chunk), :] = (
            h_ref[pl.ds(own * chunk, chunk), :]
        )

        for t in range(N_DEV - 1):
            c = lax.rem(my + 1 - t + 2 * N_DEV, N_DEV)
            rdma = pltpu.make_async_remote_copy(
                src_ref=hfull_ref.at[pl.ds(c * chunk, chunk), :],
                dst_ref=hfull_ref.at[pl.ds(c * chunk, chunk), :],
                send_sem=ag_send_sem,
                recv_sem=ag_recv_sems.at[t],
                device_id=(right,),
                device_id_type=pl.DeviceIdType.MESH,
            )
            rdma.start()
            rdma.wait()

        out_ref[...] = jnp.dot(hfull_ref[...], w2_ref[...],
                               preferred_element_type=jnp.float32)

    return pl.pallas_call(
        body,
        out_shape=jax.ShapeDtypeStruct((m, n_out), jnp.float32),
        in_specs=[
            pl.BlockSpec(memory_space=pltpu.VMEM),
            pl.BlockSpec(memory_space=pltpu.VMEM),
            pl.BlockSpec(memory_space=pltpu.VMEM),
        ],
        out_specs=pl.BlockSpec(memory_space=pltpu.VMEM),
        scratch_shapes=[
            pltpu.VMEM((m, d), jnp.float32),
            pltpu.VMEM((m, d), jnp.float32),
            pltpu.VMEM((N_DEV - 1, chunk, d), jnp.float32),
            pltpu.SemaphoreType.DMA,
            pltpu.SemaphoreType.DMA((N_DEV - 1,)),
            pltpu.SemaphoreType.DMA,
            pltpu.SemaphoreType.DMA((N_DEV - 1,)),
        ],
        compiler_params=pltpu.CompilerParams(collective_id=0),
    )(x, W1, W2)


# device time: 139422 ns/iter; 1.4364x vs baseline; 1.4364x over previous
import functools

import jax
import jax.numpy as jnp
from jax import lax
from jax.experimental import pallas as pl
from jax.experimental.pallas import tpu as pltpu

N_DEV = 8


def kernel(x, W1, W2):
    m, _ = x.shape
    d = W1.shape[1]
    n_out = W2.shape[1]
    chunk = m // N_DEV

    def body(x_ref, w1_ref, w2_ref, out_ref,
             h_ref, hc_ref, w2full_ref, oc_ref, rs_recv_ref,
             w2_send_sems, w2_recv_sems, rs_send_sems, rs_recv_sems,
             out_send_sems, out_recv_sems):
        my = lax.axis_index("i")

        barrier_sem = pltpu.get_barrier_semaphore()
        for t in range(1, N_DEV):
            peer = lax.rem(my + t, N_DEV)
            pl.semaphore_signal(
                barrier_sem, inc=1,
                device_id=(peer,), device_id_type=pl.DeviceIdType.MESH,
            )
        pl.semaphore_wait(barrier_sem, N_DEV - 1)

        w2_rdmas = []
        for t in range(N_DEV - 1):
            peer = lax.rem(my + 1 + t, N_DEV)
            rdma = pltpu.make_async_remote_copy(
                src_ref=w2_ref,
                dst_ref=w2full_ref.at[:, pl.ds(my * n_out, n_out)],
                send_sem=w2_send_sems.at[t],
                recv_sem=w2_recv_sems.at[N_DEV - 2 - t],
                device_id=(peer,),
                device_id_type=pl.DeviceIdType.MESH,
            )
            rdma.start()
            w2_rdmas.append(rdma)
        w2full_ref[:, pl.ds(my * n_out, n_out)] = w2_ref[...]

        h_ref[...] = jnp.dot(x_ref[...], w1_ref[...],
                             preferred_element_type=jnp.float32)

        rs_rdmas = []
        for t in range(N_DEV - 1):
            peer = lax.rem(my + 1 + t, N_DEV)
            rdma = pltpu.make_async_remote_copy(
                src_ref=h_ref.at[pl.ds(peer * chunk, chunk), :],
                dst_ref=rs_recv_ref.at[N_DEV - 2 - t],
                send_sem=rs_send_sems.at[t],
                recv_sem=rs_recv_sems.at[N_DEV - 2 - t],
                device_id=(peer,),
                device_id_type=pl.DeviceIdType.MESH,
            )
            rdma.start()
            rs_rdmas.append(rdma)

        acc = h_ref[pl.ds(my * chunk, chunk), :]
        for t in range(N_DEV - 1):
            rs_rdmas[t].wait_recv()
            acc = acc + rs_recv_ref[N_DEV - 2 - t]
        hc_ref[...] = acc

        for t in range(N_DEV - 1):
            w2_rdmas[t].wait_recv()
        oc_ref[...] = jnp.dot(hc_ref[...], w2full_ref[...],
                              preferred_element_type=jnp.float32)

        out_rdmas = []
        for t in range(N_DEV - 1):
            peer = lax.rem(my + 1 + t, N_DEV)
            rdma = pltpu.make_async_remote_copy(
                src_ref=oc_ref.at[:, pl.ds(peer * n_out, n_out)],
                dst_ref=out_ref.at[pl.ds(my * chunk, chunk), :],
                send_sem=out_send_sems.at[t],
                recv_sem=out_recv_sems.at[N_DEV - 2 - t],
                device_id=(peer,),
                device_id_type=pl.DeviceIdType.MESH,
            )
            rdma.start()
            out_rdmas.append(rdma)
        out_ref[pl.ds(my * chunk, chunk), :] = oc_ref[:, pl.ds(my * n_out, n_out)]

        for t in range(N_DEV - 1):
            out_rdmas[t].wait_recv()
        for rdma in w2_rdmas + rs_rdmas + out_rdmas:
            rdma.wait_send()

        @functools.partial(pl.run_scoped,
                           exit_sem=pltpu.SemaphoreType.REGULAR)
        def _(exit_sem):
            for t in range(1, N_DEV):
                peer = lax.rem(my + t, N_DEV)
                pl.semaphore_signal(
                    exit_sem, inc=1,
                    device_id=(peer,), device_id_type=pl.DeviceIdType.MESH,
                )
            pl.semaphore_wait(exit_sem, N_DEV - 1)

    return pl.pallas_call(
        body,
        out_shape=jax.ShapeDtypeStruct((m, n_out), jnp.float32),
        in_specs=[
            pl.BlockSpec(memory_space=pltpu.VMEM),
            pl.BlockSpec(memory_space=pltpu.VMEM),
            pl.BlockSpec(memory_space=pltpu.VMEM),
        ],
        out_specs=pl.BlockSpec(memory_space=pltpu.VMEM),
        scratch_shapes=[
            pltpu.VMEM((m, d), jnp.float32),
            pltpu.VMEM((chunk, d), jnp.float32),
            pltpu.VMEM((d, N_DEV * n_out), jnp.float32),
            pltpu.VMEM((chunk, N_DEV * n_out), jnp.float32),
            pltpu.VMEM((N_DEV - 1, chunk, d), jnp.float32),
            pltpu.SemaphoreType.DMA((N_DEV - 1,)),
            pltpu.SemaphoreType.DMA((N_DEV - 1,)),
            pltpu.SemaphoreType.DMA((N_DEV - 1,)),
            pltpu.SemaphoreType.DMA((N_DEV - 1,)),
            pltpu.SemaphoreType.DMA((N_DEV - 1,)),
            pltpu.SemaphoreType.DMA((N_DEV - 1,)),
        ],
        compiler_params=pltpu.CompilerParams(collective_id=0),
    )(x, W1, W2)


# device time: 83330 ns/iter; 2.4033x vs baseline; 1.6731x over previous
import functools

import jax
import jax.numpy as jnp
from jax import lax
from jax.experimental import pallas as pl
from jax.experimental.pallas import tpu as pltpu

N_DEV = 8


def kernel(x, W1, W2):
    m, _ = x.shape
    d = W1.shape[1]
    n_out = W2.shape[1]
    chunk = m // N_DEV

    def body(x_ref, w1_ref, w2_ref, out_ref,
             h_ref, hc_ref, w2full_ref, oc_ref, rs_recv_ref,
             rs_send_sems, rs_recv_sems):
        my = lax.axis_index("i")

        barrier_sem = pltpu.get_barrier_semaphore()
        for t in range(1, N_DEV):
            peer = lax.rem(my + t, N_DEV)
            pl.semaphore_signal(
                barrier_sem, inc=1,
                device_id=(peer,), device_id_type=pl.DeviceIdType.MESH,
            )
        pl.semaphore_wait(barrier_sem, N_DEV - 1)

        w2full_ref[:, pl.ds(my * n_out, n_out)] = w2_ref[...]

        h_ref[...] = jnp.dot(x_ref[...], w1_ref[...],
                             preferred_element_type=jnp.float32)

        rs_rdmas = []
        for t in range(N_DEV - 1):
            peer = lax.rem(my + 1 + t, N_DEV)
            rdma = pltpu.make_async_remote_copy(
                src_ref=h_ref.at[pl.ds(peer * chunk, chunk), :],
                dst_ref=rs_recv_ref.at[N_DEV - 2 - t],
                send_sem=rs_send_sems.at[t],
                recv_sem=rs_recv_sems.at[N_DEV - 2 - t],
                device_id=(peer,),
                device_id_type=pl.DeviceIdType.MESH,
            )
            rdma.start()
            rs_rdmas.append(rdma)

        acc = h_ref[pl.ds(my * chunk, chunk), :]
        for t in range(N_DEV - 1):
            rs_rdmas[t].wait_recv()
            acc = acc + rs_recv_ref[N_DEV - 2 - t]
        hc_ref[...] = acc

        oc_ref[...] = jnp.dot(hc_ref[...], w2full_ref[...],
                              preferred_element_type=jnp.float32)

        out_ref[...] = jnp.zeros((m, n_out), jnp.float32)
        out_ref[pl.ds(my * chunk, chunk), :] = oc_ref[:, pl.ds(my * n_out, n_out)]

        for rdma in rs_rdmas:
            rdma.wait_send()

        @functools.partial(pl.run_scoped,
                           exit_sem=pltpu.SemaphoreType.REGULAR)
        def _(exit_sem):
            for t in range(1, N_DEV):
                peer = lax.rem(my + t, N_DEV)
                pl.semaphore_signal(
                    exit_sem, inc=1,
                    device_id=(peer,), device_id_type=pl.DeviceIdType.MESH,
                )
            pl.semaphore_wait(exit_sem, N_DEV - 1)

    return pl.pallas_call(
        body,
        out_shape=jax.ShapeDtypeStruct((m, n_out), jnp.float32),
        in_specs=[
            pl.BlockSpec(memory_space=pltpu.VMEM),
            pl.BlockSpec(memory_space=pltpu.VMEM),
            pl.BlockSpec(memory_space=pltpu.VMEM),
        ],
        out_specs=pl.BlockSpec(memory_space=pltpu.VMEM),
        scratch_shapes=[
            pltpu.VMEM((m, d), jnp.float32),
            pltpu.VMEM((chunk, d), jnp.float32),
            pltpu.VMEM((d, N_DEV * n_out), jnp.float32),
            pltpu.VMEM((chunk, N_DEV * n_out), jnp.float32),
            pltpu.VMEM((N_DEV - 1, chunk, d), jnp.float32),
            pltpu.SemaphoreType.DMA((N_DEV - 1,)),
            pltpu.SemaphoreType.DMA((N_DEV - 1,)),
        ],
        compiler_params=pltpu.CompilerParams(collective_id=0),
    )(x, W1, W2)


# device time: 75946 ns/iter; 2.6369x vs baseline; 1.0972x over previous
import functools

import jax
import jax.numpy as jnp
from jax import lax
from jax.experimental import pallas as pl
from jax.experimental.pallas import tpu as pltpu

N_DEV = 8


def kernel(x, W1, W2):
    m, _ = x.shape
    d = W1.shape[1]
    n_out = W2.shape[1]
    chunk = m // N_DEV

    def body(x_ref, w1_ref, w2_ref, out_ref,
             h_ref, hbf_ref, hc_ref, w2bf_ref, w2full_ref,
             oc_ref, ocbf_ref, rs_recv_ref, out_recv_ref,
             w2_send_sems, w2_recv_sems, rs_send_sems, rs_recv_sems,
             out_send_sems, out_recv_sems):
        my = lax.axis_index("i")

        barrier_sem = pltpu.get_barrier_semaphore()
        for t in range(1, N_DEV):
            peer = lax.rem(my + t, N_DEV)
            pl.semaphore_signal(
                barrier_sem, inc=1,
                device_id=(peer,), device_id_type=pl.DeviceIdType.MESH,
            )
        pl.semaphore_wait(barrier_sem, N_DEV - 1)

        w2bf_ref[...] = w2_ref[...].astype(jnp.bfloat16)
        w2_rdmas = []
        for t in range(N_DEV - 1):
            peer = lax.rem(my + 1 + t, N_DEV)
            rdma = pltpu.make_async_remote_copy(
                src_ref=w2bf_ref,
                dst_ref=w2full_ref.at[:, pl.ds(my * n_out, n_out)],
                send_sem=w2_send_sems.at[t],
                recv_sem=w2_recv_sems.at[N_DEV - 2 - t],
                device_id=(peer,),
                device_id_type=pl.DeviceIdType.MESH,
            )
            rdma.start()
            w2_rdmas.append(rdma)
        w2full_ref[:, pl.ds(my * n_out, n_out)] = w2bf_ref[...]

        h_ref[...] = jnp.dot(x_ref[...], w1_ref[...],
                             preferred_element_type=jnp.float32)
        hbf_ref[...] = h_ref[...].astype(jnp.bfloat16)

        rs_rdmas = []
        for t in range(N_DEV - 1):
            peer = lax.rem(my + 1 + t, N_DEV)
            rdma = pltpu.make_async_remote_copy(
                src_ref=hbf_ref.at[pl.ds(peer * chunk, chunk), :],
                dst_ref=rs_recv_ref.at[N_DEV - 2 - t],
                send_sem=rs_send_sems.at[t],
                recv_sem=rs_recv_sems.at[N_DEV - 2 - t],
                device_id=(peer,),
                device_id_type=pl.DeviceIdType.MESH,
            )
            rdma.start()
            rs_rdmas.append(rdma)

        acc = h_ref[pl.ds(my * chunk, chunk), :]
        for t in range(N_DEV - 1):
            rs_rdmas[t].wait_recv()
            acc = acc + rs_recv_ref[N_DEV - 2 - t].astype(jnp.float32)
        hc_ref[...] = acc.astype(jnp.bfloat16)

        for t in range(N_DEV - 1):
            w2_rdmas[t].wait_recv()
        oc_ref[...] = jnp.dot(hc_ref[...], w2full_ref[...],
                              preferred_element_type=jnp.float32)
        ocbf_ref[...] = oc_ref[...].astype(jnp.bfloat16)

        out_rdmas = []
        for t in range(N_DEV - 1):
            peer = lax.rem(my + 1 + t, N_DEV)
            rdma = pltpu.make_async_remote_copy(
                src_ref=ocbf_ref.at[:, pl.ds(peer * n_out, n_out)],
                dst_ref=out_recv_ref.at[N_DEV - 2 - t],
                send_sem=out_send_sems.at[t],
                recv_sem=out_recv_sems.at[N_DEV - 2 - t],
                device_id=(peer,),
                device_id_type=pl.DeviceIdType.MESH,
            )
            rdma.start()
            out_rdmas.append(rdma)
        out_ref[pl.ds(my * chunk, chunk), :] = oc_ref[:, pl.ds(my * n_out, n_out)]

        for t in range(N_DEV - 1):
            out_rdmas[t].wait_recv()
            src = lax.rem(my + N_DEV - 1 - t, N_DEV)
            out_ref[pl.ds(src * chunk, chunk), :] = (
                out_recv_ref[N_DEV - 2 - t].astype(jnp.float32)
            )

        for rdma in w2_rdmas + rs_rdmas + out_rdmas:
            rdma.wait_send()

        @functools.partial(pl.run_scoped,
                           exit_sem=pltpu.SemaphoreType.REGULAR)
        def _(exit_sem):
            for t in range(1, N_DEV):
                peer = lax.rem(my + t, N_DEV)
                pl.semaphore_signal(
                    exit_sem, inc=1,
                    device_id=(peer,), device_id_type=pl.DeviceIdType.MESH,
                )
            pl.semaphore_wait(exit_sem, N_DEV - 1)

    return pl.pallas_call(
        body,
        out_shape=jax.ShapeDtypeStruct((m, n_out), jnp.float32),
        in_specs=[
            pl.BlockSpec(memory_space=pltpu.VMEM),
            pl.BlockSpec(memory_space=pltpu.VMEM),
            pl.BlockSpec(memory_space=pltpu.VMEM),
        ],
        out_specs=pl.BlockSpec(memory_space=pltpu.VMEM),
        scratch_shapes=[
            pltpu.VMEM((m, d), jnp.float32),
            pltpu.VMEM((m, d), jnp.bfloat16),
            pltpu.VMEM((chunk, d), jnp.bfloat16),
            pltpu.VMEM((d, n_out), jnp.bfloat16),
            pltpu.VMEM((d, N_DEV * n_out), jnp.bfloat16),
            pltpu.VMEM((chunk, N_DEV * n_out), jnp.float32),
            pltpu.VMEM((chunk, N_DEV * n_out), jnp.bfloat16),
            pltpu.VMEM((N_DEV - 1, chunk, d), jnp.bfloat16),
            pltpu.VMEM((N_DEV - 1, chunk, n_out), jnp.bfloat16),
            pltpu.SemaphoreType.DMA((N_DEV - 1,)),
            pltpu.SemaphoreType.DMA((N_DEV - 1,)),
            pltpu.SemaphoreType.DMA((N_DEV - 1,)),
            pltpu.SemaphoreType.DMA((N_DEV - 1,)),
            pltpu.SemaphoreType.DMA((N_DEV - 1,)),
            pltpu.SemaphoreType.DMA((N_DEV - 1,)),
        ],
        compiler_params=pltpu.CompilerParams(collective_id=0),
    )(x, W1, W2)
